# Initial kernel scaffold; baseline (speedup 1.0000x reference)
import jax
import jax.numpy as jnp
from jax.experimental import pallas as pl


def kernel(x, t_vertex, neighbor_index, W_mlp, b_mlp, W_out, bias):
    raise NotImplementedError("write your pallas kernel here")


# speedup vs baseline: 1.9828x; 1.9828x over previous
import functools

import jax
import jax.numpy as jnp
from jax import lax
from jax.experimental import pallas as pl
from jax.experimental.pallas import tpu as pltpu
from jax.experimental.pallas import tpu_sc as plsc

N_PTS = 10000
IN_C = 128
K = 16
H = 16
OUT_C = 32

NC, NS = 2, 16
NW = NC * NS
CH = 128
NCH = 40
BPW = NCH * CH
B_PAD = NW * BPW


def _sc_gather(x2d, idx3):
    mesh = plsc.VectorSubcoreMesh(core_axis_name="c", subcore_axis_name="s")

    @functools.partial(
        pl.kernel,
        mesh=mesh,
        out_type=jax.ShapeDtypeStruct((B_PAD, IN_C), jnp.float32),
        scratch_types=[
            pltpu.VMEM((NCH, CH), jnp.int32),
            pltpu.VMEM((CH, IN_C), jnp.float32),
            pltpu.SemaphoreType.DMA,
        ],
    )
    def gather_kernel(x_hbm, idx_hbm, out_hbm, idx_v, rows_v, sem):
        wid = lax.axis_index("s") * NC + lax.axis_index("c")
        base = wid * BPW
        pltpu.sync_copy(idx_hbm.at[wid], idx_v)

        def body(ch, carry):
            pltpu.async_copy(x_hbm.at[idx_v.at[ch]], rows_v, sem).wait()
            pltpu.sync_copy(rows_v, out_hbm.at[pl.ds(base + ch * CH, CH)])
            return carry

        lax.fori_loop(0, NCH, body, 0)

    return gather_kernel(x2d, idx3)


PB = 400
GRID = N_PTS // PB


def _tc_body(xg_ref, wm_ref, bm_ref, wr_ref, bias_ref, out_ref):
    i = pl.program_id(0)
    wm = wm_ref[...]
    bm = bm_ref[...]

    xk = [xg_ref[:, k, :] for k in range(K)]
    lk = [
        lax.dot_general(xk[k], wm, (((1,), (1,)), ((), ())),
                        preferred_element_type=jnp.float32)
        for k in range(K)
    ]
    l0 = lk[0]
    agg = [None] * H
    for k in range(K):
        dk = lk[k] - l0 + bm
        m = jnp.max(dk, axis=1, keepdims=True)
        ek = jnp.exp(dk - m)
        sk = jnp.sum(ek, axis=1, keepdims=True)
        xks = xk[k] / sk
        for h in range(H):
            t = ek[:, h:h + 1] * xks
            agg[h] = t if agg[h] is None else agg[h] + t
    agg2 = jnp.concatenate(agg, axis=1)
    out = lax.dot_general(agg2, wr_ref[...], (((1,), (0,)), ((), ())),
                          preferred_element_type=jnp.float32)
    out = out + bias_ref[...]
    gp = i * PB + lax.broadcasted_iota(jnp.int32, (PB, OUT_C), 0)
    out = jnp.where(gp == N_PTS - 1, 0.0, out)
    out_ref[...] = jnp.maximum(out, 0.0)


def _tc_compute(xg3, W_mlp, bm2, wr, bias2):
    return pl.pallas_call(
        _tc_body,
        grid=(GRID,),
        in_specs=[
            pl.BlockSpec((PB, K, IN_C), lambda i: (i, 0, 0)),
            pl.BlockSpec((H, IN_C), lambda i: (0, 0)),
            pl.BlockSpec((1, H), lambda i: (0, 0)),
            pl.BlockSpec((H * IN_C, OUT_C), lambda i: (0, 0)),
            pl.BlockSpec((1, OUT_C), lambda i: (0, 0)),
        ],
        out_specs=pl.BlockSpec((PB, OUT_C), lambda i: (i, 0)),
        out_shape=jax.ShapeDtypeStruct((N_PTS, OUT_C), jnp.float32),
    )(xg3, W_mlp, bm2, wr, bias2)


def kernel(x, t_vertex, neighbor_index, W_mlp, b_mlp, W_out, bias):
    x2d = x[0]
    idx = neighbor_index[0].astype(jnp.int32).reshape(-1)
    idx = jnp.pad(idx, (0, B_PAD - N_PTS * K))
    idx3 = idx.reshape(NW, NCH, CH)

    xg = _sc_gather(x2d, idx3)
    xg3 = xg.reshape(B_PAD // K, K, IN_C)

    wr = W_out.reshape(H, OUT_C, IN_C).transpose(0, 2, 1).reshape(H * IN_C, OUT_C)

    out = _tc_compute(xg3, W_mlp, b_mlp.reshape(1, H), wr, bias.reshape(1, OUT_C))
    return out[None]


# speedup vs baseline: 3.9633x; 1.9988x over previous
import functools

import jax
import jax.numpy as jnp
from jax import lax
from jax.experimental import pallas as pl
from jax.experimental.pallas import tpu as pltpu
from jax.experimental.pallas import tpu_sc as plsc

N_PTS = 10000
IN_C = 128
K = 16
H = 16
OUT_C = 32

NC, NS = 2, 16
NW = NC * NS
CH = 128
NCH = 40
BPW = NCH * CH
B_PAD = NW * BPW
NBUF = 4


def _sc_gather(x2d, idx3):
    mesh = plsc.VectorSubcoreMesh(core_axis_name="c", subcore_axis_name="s")

    @functools.partial(
        pl.kernel,
        mesh=mesh,
        out_type=jax.ShapeDtypeStruct((B_PAD, IN_C), jnp.float32),
        scratch_types=[
            pltpu.VMEM((NCH, CH), jnp.int32),
        ] + [pltpu.VMEM((CH, IN_C), jnp.float32) for _ in range(NBUF)]
          + [pltpu.SemaphoreType.DMA for _ in range(NBUF)],
    )
    def gather_kernel(x_hbm, idx_hbm, out_hbm, idx_v, *bufs_and_sems):
        rows = bufs_and_sems[:NBUF]
        sems = bufs_and_sems[NBUF:]
        wid = lax.axis_index("s") * NC + lax.axis_index("c")
        base = wid * BPW
        pltpu.sync_copy(idx_hbm.at[wid], idx_v)

        def start(c, b):
            pltpu.async_copy(x_hbm.at[idx_v.at[c]], rows[b], sems[b])

        for b in range(NBUF):
            start(b, b)

        def body(g, carry):
            for b in range(NBUF):
                c = g * NBUF + b
                pltpu.make_async_copy(x_hbm.at[idx_v.at[c]], rows[b],
                                      sems[b]).wait()
                pltpu.sync_copy(rows[b], out_hbm.at[pl.ds(base + c * CH, CH)])

                @pl.when(g < NCH // NBUF - 1)
                def _():
                    start(c + NBUF, b)
            return carry

        lax.fori_loop(0, NCH // NBUF, body, 0)

    return gather_kernel(x2d, idx3)


PB = 400
GRID = N_PTS // PB


def _tc_body(xg_ref, wm_ref, bm_ref, wr_ref, bias_ref, out_ref):
    i = pl.program_id(0)
    wm = wm_ref[...]
    bm = bm_ref[...]

    lk = [
        lax.dot_general(xg_ref[:, k, :], wm, (((1,), (1,)), ((), ())),
                        preferred_element_type=jnp.float32)
        for k in range(K)
    ]
    l0 = lk[0]
    ekn = []
    for k in range(K):
        dk = lk[k] - l0 + bm
        m = jnp.max(dk, axis=1, keepdims=True)
        ek = jnp.exp(dk - m)
        sk = jnp.sum(ek, axis=1, keepdims=True)
        ekn.append(ek / sk)

    rowid = lax.broadcasted_iota(jnp.int32, (H, H * IN_C), 0)
    colh = lax.shift_right_logical(
        lax.broadcasted_iota(jnp.int32, (H, H * IN_C), 1), 7)
    S = jnp.where(rowid == colh, 1.0, 0.0).astype(jnp.float32)

    aggs = []
    for h in range(H):
        Sh = S[:, h * IN_C:(h + 1) * IN_C]
        acc = None
        for k in range(K):
            ew = lax.dot_general(ekn[k], Sh, (((1,), (0,)), ((), ())),
                                 preferred_element_type=jnp.float32)
            t = ew * xg_ref[:, k, :]
            acc = t if acc is None else acc + t
        aggs.append(acc)
    agg2 = jnp.concatenate(aggs, axis=1)
    out = lax.dot_general(agg2, wr_ref[...], (((1,), (0,)), ((), ())),
                          preferred_element_type=jnp.float32)
    out = out + bias_ref[...]
    gp = i * PB + lax.broadcasted_iota(jnp.int32, (PB, OUT_C), 0)
    out = jnp.where(gp == N_PTS - 1, 0.0, out)
    out_ref[...] = jnp.maximum(out, 0.0)


def _tc_compute(xg3, W_mlp, bm2, wr, bias2):
    return pl.pallas_call(
        _tc_body,
        grid=(GRID,),
        in_specs=[
            pl.BlockSpec((PB, K, IN_C), lambda i: (i, 0, 0)),
            pl.BlockSpec((H, IN_C), lambda i: (0, 0)),
            pl.BlockSpec((1, H), lambda i: (0, 0)),
            pl.BlockSpec((H * IN_C, OUT_C), lambda i: (0, 0)),
            pl.BlockSpec((1, OUT_C), lambda i: (0, 0)),
        ],
        out_specs=pl.BlockSpec((PB, OUT_C), lambda i: (i, 0)),
        out_shape=jax.ShapeDtypeStruct((N_PTS, OUT_C), jnp.float32),
    )(xg3, W_mlp, bm2, wr, bias2)


def kernel(x, t_vertex, neighbor_index, W_mlp, b_mlp, W_out, bias):
    x2d = x[0]
    idx = neighbor_index[0].astype(jnp.int32).reshape(-1)
    idx = jnp.pad(idx, (0, B_PAD - N_PTS * K))
    idx3 = idx.reshape(NW, NCH, CH)

    xg = _sc_gather(x2d, idx3)
    xg3 = xg.reshape(B_PAD // K, K, IN_C)

    wr = W_out.reshape(H, OUT_C, IN_C).transpose(0, 2, 1).reshape(H * IN_C, OUT_C)

    out = _tc_compute(xg3, W_mlp, b_mlp.reshape(1, H), wr, bias.reshape(1, OUT_C))
    return out[None]
